# baseline (speedup 1.0000x reference)
import jax
import jax.numpy as jnp
from jax.experimental import pallas as pl
from jax.experimental.pallas import tpu as pltpu

_B, _N = 256, 512
_EMB, _VIS, _KPT = 128, 1, 17
_TOKEN_DIM = 128
_ROWS = 1024


def _body(mask_ref, emb_ref, vis_ref, bbox_ref, kpt_ref,
          wemb_ref, wvis_ref, wbbox_ref, wkpt_ref, ab_ref, sb_ref, out_ref):
    dn = (((1,), (1,)), ((), ()))
    acc = jax.lax.dot_general(emb_ref[...], wemb_ref[...], dn,
                              preferred_element_type=jnp.float32)
    acc += jax.lax.dot_general(kpt_ref[...], wkpt_ref[...], dn,
                               preferred_element_type=jnp.float32)
    acc += jax.lax.dot_general(bbox_ref[...], wbbox_ref[...], dn,
                               preferred_element_type=jnp.float32)
    acc += vis_ref[...] * wvis_ref[...]
    acc += ab_ref[...] + sb_ref[...]
    out_ref[...] = acc * mask_ref[...]


def kernel(feats_masks, embeddings, visibility_scores, bbox_ltwh,
           keypoints_xyc, app_W, app_b, st_W, st_b):
    m = _B * _N
    mask = feats_masks.reshape(m, 1).astype(jnp.float32)
    emb = embeddings.reshape(m, _EMB)
    vis = visibility_scores.reshape(m, _VIS)
    bbox = bbox_ltwh.reshape(m, 4)
    kpt = keypoints_xyc.reshape(m, _KPT * 3)
    wemb = app_W[:, :_EMB]
    wvis = app_W[:, _EMB].reshape(1, _TOKEN_DIM)
    wbbox = st_W[:, :4]
    wkpt = st_W[:, 4:]
    ab = app_b.reshape(1, _TOKEN_DIM)
    sb = st_b.reshape(1, _TOKEN_DIM)

    grid = (m // _ROWS,)
    row = lambda i: (i, 0)
    rep = lambda i: (0, 0)
    out = pl.pallas_call(
        _body,
        grid=grid,
        in_specs=[
            pl.BlockSpec((_ROWS, 1), row),
            pl.BlockSpec((_ROWS, _EMB), row),
            pl.BlockSpec((_ROWS, _VIS), row),
            pl.BlockSpec((_ROWS, 4), row),
            pl.BlockSpec((_ROWS, _KPT * 3), row),
            pl.BlockSpec((_TOKEN_DIM, _EMB), rep),
            pl.BlockSpec((1, _TOKEN_DIM), rep),
            pl.BlockSpec((_TOKEN_DIM, 4), rep),
            pl.BlockSpec((_TOKEN_DIM, _KPT * 3), rep),
            pl.BlockSpec((1, _TOKEN_DIM), rep),
            pl.BlockSpec((1, _TOKEN_DIM), rep),
        ],
        out_specs=pl.BlockSpec((_ROWS, _TOKEN_DIM), row),
        out_shape=jax.ShapeDtypeStruct((m, _TOKEN_DIM), jnp.float32),
        compiler_params=pltpu.CompilerParams(
            dimension_semantics=("arbitrary",),
        ),
    )(mask, emb, vis, bbox, kpt, wemb, wvis, wbbox, wkpt, ab, sb)
    return out.reshape(_B, _N, _TOKEN_DIM)
